# baseline (speedup 1.0000x reference)
import functools

import jax
import jax.numpy as jnp
from jax import lax
from jax.experimental import pallas as pl
from jax.experimental.pallas import tpu as pltpu, tpu_sc as plsc

N = 50000
E = 800000
D = 64
V = 100000
G = 128

NC = 2
NS = 16
NW = NC * NS

NP = 51200
RPT = NP // NS
EB = NP // NW
CH = 128
NCHUNK = 200
NBLK = 25
EPT = CH * NCHUNK
EP = EPT * NW
NBUF = 4
TRASH = N

_SC_PARAMS = pltpu.CompilerParams(use_tc_tiling_on_sc=False)
_mesh = functools.partial(plsc.VectorSubcoreMesh,
                          core_axis_name="c", subcore_axis_name="s")


@functools.partial(
    pl.kernel, mesh=_mesh(),
    out_type=(jax.ShapeDtypeStruct((NP, D), jnp.float32),
              jax.ShapeDtypeStruct((NC, NP, 8), jnp.float32)),
    compiler_params=_SC_PARAMS,
    scratch_types=[
        pltpu.VMEM((EB,), jnp.int32),
        pltpu.VMEM((EB // 2, D), jnp.float32),
        pltpu.VMEM((NCHUNK, CH), jnp.int32),
        pltpu.VMEM((CH, 8), jnp.float32),
        pltpu.VMEM_SHARED((NP, 8), jnp.float32),
        pltpu.SemaphoreType.DMA,
    ],
)
def _sc_emb_deg(emb_hbm, xp_hbm, dstb_hbm, ones_hbm, z16_hbm,
                h_hbm, degp_hbm,
                xidx_v, hrows_v, dst_v, ones_v, acc16, sem):
    cid = lax.axis_index("c")
    sid = lax.axis_index("s")
    wid = sid * NC + cid
    base = wid * EB

    pltpu.sync_copy(xp_hbm.at[pl.ds(base, EB)], xidx_v)
    for half in range(2):
        off = half * (EB // 2)
        pltpu.async_copy(emb_hbm.at[xidx_v.at[pl.ds(off, EB // 2)]],
                         hrows_v, sem).wait()
        pltpu.sync_copy(hrows_v, h_hbm.at[pl.ds(base + off, EB // 2)])

    pltpu.sync_copy(ones_hbm, ones_v)
    pltpu.sync_copy(dstb_hbm.at[wid], dst_v)
    pltpu.sync_copy(z16_hbm, acc16.at[pl.ds(sid * RPT, RPT)])
    plsc.subcore_barrier()

    def deg_body(j, carry):
        pltpu.sync_copy(ones_v, acc16.at[dst_v.at[j]], add=True)
        return carry

    lax.fori_loop(0, NCHUNK, deg_body, 0)
    plsc.subcore_barrier()
    pltpu.sync_copy(acc16.at[pl.ds(sid * RPT, RPT)],
                    degp_hbm.at[cid].at[pl.ds(sid * RPT, RPT)])


@functools.partial(
    pl.kernel, mesh=_mesh(),
    out_type=(jax.ShapeDtypeStruct((NC, NP, 32), jnp.float32),
              jax.ShapeDtypeStruct((NC, NP, 32), jnp.float32)),
    compiler_params=_SC_PARAMS,
    scratch_types=[
        pltpu.VMEM((2, 8, 2, CH), jnp.int32),
        pltpu.VMEM((NBUF, CH, 32), jnp.float32),
        pltpu.VMEM_SHARED((NP, 32), jnp.float32),
        pltpu.SemaphoreType.DMA,
        pltpu.SemaphoreType.DMA,
        pltpu.SemaphoreType.DMA,
        pltpu.SemaphoreType.DMA,
        pltpu.SemaphoreType.DMA,
        pltpu.SemaphoreType.DMA,
        pltpu.SemaphoreType.DMA,
        pltpu.SemaphoreType.DMA,
        pltpu.SemaphoreType.DMA,
    ],
)
def _sc_edge_pass(hsa_hbm, hsb_hbm, edges_hbm, z32_hbm,
                  oa_hbm, ob_hbm,
                  iblk, rbuf, acc, isem,
                  g0, g1, g2, g3, t0, t1, t2, t3):
    cid = lax.axis_index("c")
    sid = lax.axis_index("s")
    wid = sid * NC + cid
    gsems = (g0, g1, g2, g3)
    ssems = (t0, t1, t2, t3)

    for p in range(2):
        hs_hbm = hsa_hbm if p == 0 else hsb_hbm
        out_hbm = oa_hbm if p == 0 else ob_hbm

        pltpu.sync_copy(z32_hbm, acc.at[pl.ds(sid * RPT, RPT)])
        pltpu.sync_copy(edges_hbm.at[wid, 0], iblk.at[0])
        plsc.subcore_barrier()

        for b in range(2):
            pltpu.async_copy(hs_hbm.at[iblk.at[0, b, 0]], rbuf.at[b], gsems[b])
        pltpu.async_copy(edges_hbm.at[wid, 1], iblk.at[1], isem)

        def outer(s, carry):
            cur = s % 2
            nxtb = (s + 1) % 2
            for j in range(8):
                b = j % NBUF
                bn = (j + 2) % NBUF
                g = s * 8 + j
                pltpu.make_async_copy(hs_hbm.at[iblk.at[cur, j, 0]],
                                      rbuf.at[b], gsems[b]).wait()
                pltpu.async_copy(rbuf.at[b], acc.at[iblk.at[cur, j, 1]],
                                 ssems[b], add=True)
                if j == 4:
                    @pl.when(s + 1 < NBLK)
                    def _():
                        pltpu.make_async_copy(edges_hbm.at[wid, 0],
                                              iblk.at[nxtb], isem).wait()
                @pl.when(g >= 2)
                def _():
                    pltpu.make_async_copy(rbuf.at[bn],
                                          acc.at[iblk.at[cur, j, 1]],
                                          ssems[bn]).wait()
                if j < 6:
                    @pl.when(g + 2 < NCHUNK)
                    def _():
                        pltpu.async_copy(hs_hbm.at[iblk.at[cur, j + 2, 0]],
                                         rbuf.at[bn], gsems[bn])
                else:
                    @pl.when(g + 2 < NCHUNK)
                    def _():
                        pltpu.async_copy(hs_hbm.at[iblk.at[nxtb, j - 6, 0]],
                                         rbuf.at[bn], gsems[bn])
            @pl.when(s + 2 < NBLK)
            def _():
                pltpu.async_copy(edges_hbm.at[wid, s + 2], iblk.at[cur], isem)
            return carry

        lax.fori_loop(0, NBLK, outer, 0)
        for b in (2, 3):
            pltpu.make_async_copy(rbuf.at[b], acc.at[iblk.at[0, 0, 1]],
                                  ssems[b]).wait()
        plsc.subcore_barrier()
        pltpu.sync_copy(acc.at[pl.ds(sid * RPT, RPT)],
                        out_hbm.at[cid].at[pl.ds(sid * RPT, RPT)])
        plsc.subcore_barrier()


def _tc_scale_kernel(d0_ref, d1_ref, h_ref, hsa_ref, hsb_ref, dinv_ref):
    deg = d0_ref[0] + d1_ref[0] + 1.0
    dinv8 = lax.rsqrt(jnp.maximum(deg, 1.0))
    dinv16 = jnp.concatenate([dinv8, dinv8], axis=1)
    dinv32 = jnp.concatenate([dinv16, dinv16], axis=1)
    dinv64 = jnp.concatenate([dinv32, dinv32], axis=1)
    hs = h_ref[...] * dinv64
    hsa_ref[...] = hs[:, :32]
    hsb_ref[...] = hs[:, 32:]
    dinv_ref[...] = dinv32


def _tc_scale(degp, h):
    grid = NP // RPT
    return pl.pallas_call(
        _tc_scale_kernel,
        grid=(grid,),
        in_specs=[
            pl.BlockSpec((1, RPT, 8), lambda i: (0, i, 0)),
            pl.BlockSpec((1, RPT, 8), lambda i: (1, i, 0)),
            pl.BlockSpec((RPT, D), lambda i: (i, 0)),
        ],
        out_specs=[
            pl.BlockSpec((RPT, 32), lambda i: (i, 0)),
            pl.BlockSpec((RPT, 32), lambda i: (i, 0)),
            pl.BlockSpec((RPT, 32), lambda i: (i, 0)),
        ],
        out_shape=[
            jax.ShapeDtypeStruct((NP, 32), jnp.float32),
            jax.ShapeDtypeStruct((NP, 32), jnp.float32),
            jax.ShapeDtypeStruct((NP, 32), jnp.float32),
        ],
    )(degp, degp, h)


def _tc_layer_kernel(a0_ref, a1_ref, b0_ref, b1_ref, hsa_ref, hsb_ref,
                     dinv_ref, w_ref, brow_ref, hs1a_ref, hs1b_ref):
    dinv = dinv_ref[...]
    ta = dinv * (a0_ref[0] + a1_ref[0] + hsa_ref[...])
    tb = dinv * (b0_ref[0] + b1_ref[0] + hsb_ref[...])
    t = jnp.concatenate([ta, tb], axis=1)
    h1 = jnp.maximum(jnp.dot(t, w_ref[...],
                             preferred_element_type=jnp.float32)
                     + brow_ref[...], 0.0)
    hs1a_ref[...] = h1[:, :32] * dinv
    hs1b_ref[...] = h1[:, 32:] * dinv


def _tc_layer(oa, ob, hsa, hsb, dinv, W, b):
    grid = NP // RPT
    row = pl.BlockSpec((RPT, 32), lambda i: (i, 0))
    part = pl.BlockSpec((1, RPT, 32), lambda i: (0, i, 0))
    part1 = pl.BlockSpec((1, RPT, 32), lambda i: (1, i, 0))
    return pl.pallas_call(
        _tc_layer_kernel,
        grid=(grid,),
        in_specs=[part, part1, part, part1, row, row, row,
                  pl.BlockSpec((D, D), lambda i: (0, 0)),
                  pl.BlockSpec((1, D), lambda i: (0, 0))],
        out_specs=[row, row],
        out_shape=[
            jax.ShapeDtypeStruct((NP, 32), jnp.float32),
            jax.ShapeDtypeStruct((NP, 32), jnp.float32),
        ],
    )(oa, oa, ob, ob, hsa, hsb, dinv, W, b.reshape(1, D))


def _tc_final_kernel(a0_ref, a1_ref, b0_ref, b1_ref, hsa_ref, hsb_ref,
                     dinv_ref, w_ref, brow_ref, batch_ref, wc_ref, bc_ref,
                     out_ref, psum, pcnt):
    i = pl.program_id(0)
    dinv = dinv_ref[...]
    ta = dinv * (a0_ref[0] + a1_ref[0] + hsa_ref[...])
    tb = dinv * (b0_ref[0] + b1_ref[0] + hsb_ref[...])
    t = jnp.concatenate([ta, tb], axis=1)
    h2 = jnp.maximum(jnp.dot(t, w_ref[...],
                             preferred_element_type=jnp.float32)
                     + brow_ref[...], 0.0)
    gid = batch_ref[0]
    gids = jax.lax.broadcast_in_dim(gid, (G, RPT), (0, 1))
    rows = jax.lax.broadcasted_iota(jnp.int32, (G, RPT), 0)
    onehot = jnp.where(gids == rows, 1.0, 0.0)

    @pl.when(i == 0)
    def _():
        psum[...] = jnp.zeros_like(psum)
        pcnt[...] = jnp.zeros_like(pcnt)

    psum[...] += jnp.dot(onehot, h2, preferred_element_type=jnp.float32)
    pcnt[...] += jnp.dot(onehot, jnp.ones((RPT, D), jnp.float32),
                         preferred_element_type=jnp.float32)

    @pl.when(i == pl.num_programs(0) - 1)
    def _():
        pooled = psum[...] / jnp.maximum(pcnt[...], 1.0)
        out_ref[...] = (jnp.dot(pooled, wc_ref[...],
                                preferred_element_type=jnp.float32)
                        + bc_ref[...])


def _tc_final(oa, ob, hsa, hsb, dinv, W, b, batch3, Wc, bc):
    grid = NP // RPT
    row = pl.BlockSpec((RPT, 32), lambda i: (i, 0))
    part = pl.BlockSpec((1, RPT, 32), lambda i: (0, i, 0))
    part1 = pl.BlockSpec((1, RPT, 32), lambda i: (1, i, 0))
    return pl.pallas_call(
        _tc_final_kernel,
        grid=(grid,),
        in_specs=[part, part1, part, part1, row, row, row,
                  pl.BlockSpec((D, D), lambda i: (0, 0)),
                  pl.BlockSpec((1, D), lambda i: (0, 0)),
                  pl.BlockSpec((1, 1, RPT), lambda i: (i, 0, 0)),
                  pl.BlockSpec((D, 2), lambda i: (0, 0)),
                  pl.BlockSpec((1, 2), lambda i: (0, 0))],
        out_specs=pl.BlockSpec((G, 2), lambda i: (0, 0)),
        out_shape=jax.ShapeDtypeStruct((G, 2), jnp.float32),
        scratch_shapes=[pltpu.VMEM((G, D), jnp.float32),
                        pltpu.VMEM((G, D), jnp.float32)],
    )(oa, oa, ob, ob, hsa, hsb, dinv, W, b.reshape(1, D), batch3,
      Wc, bc.reshape(1, 2))


def kernel(x, edge_index, batch, emb, W1, b1, W2, b2, Wc, bc):
    xp = jnp.concatenate([x, jnp.zeros((NP - N,), jnp.int32)])
    src_pad = jnp.concatenate(
        [edge_index[0], jnp.zeros((EP - E,), jnp.int32)])
    dst_pad = jnp.concatenate(
        [edge_index[1], jnp.full((EP - E,), TRASH, jnp.int32)])
    edges5 = jnp.concatenate(
        [src_pad.reshape(NW, NBLK, 8, 1, CH),
         dst_pad.reshape(NW, NBLK, 8, 1, CH)], axis=3)
    dstb = dst_pad.reshape(NW, NCHUNK, CH)
    batch3 = jnp.concatenate(
        [batch, jnp.full((NP - N,), G, jnp.int32)]
    ).reshape(NP // RPT, 1, RPT)

    ones16 = jnp.ones((CH, 8), jnp.float32)
    z16 = jnp.zeros((RPT, 8), jnp.float32)
    z32 = jnp.zeros((RPT, 32), jnp.float32)

    h, degp = _sc_emb_deg(emb, xp, dstb, ones16, z16)
    hsa, hsb, dinv = _tc_scale(degp, h)
    oa1, ob1 = _sc_edge_pass(hsa, hsb, edges5, z32)
    hs1a, hs1b = _tc_layer(oa1, ob1, hsa, hsb, dinv, W1, b1)
    oa2, ob2 = _sc_edge_pass(hs1a, hs1b, edges5, z32)
    return _tc_final(oa2, ob2, hs1a, hs1b, dinv, W2, b2, batch3, Wc, bc)


# speedup vs baseline: 1.1083x; 1.1083x over previous
import functools

import jax
import jax.numpy as jnp
from jax import lax
from jax.experimental import pallas as pl
from jax.experimental.pallas import tpu as pltpu, tpu_sc as plsc

N = 50000
E = 800000
D = 64
V = 100000
G = 128

NC = 2
NS = 16
NW = NC * NS

NP = 51200
RPT = NP // NS
EB = NP // NW
CH = 128
NCHUNK = 200
NBLK = 25
EPT = CH * NCHUNK
EP = EPT * NW
NBUF = 4
NBF = 38
NBS = 12
NBT = NBF + NBS
CF = 1
TRASH = N

_SC_PARAMS = pltpu.CompilerParams(use_tc_tiling_on_sc=False)
_mesh = functools.partial(plsc.VectorSubcoreMesh,
                          core_axis_name="c", subcore_axis_name="s")


@functools.partial(
    pl.kernel, mesh=_mesh(),
    out_type=(jax.ShapeDtypeStruct((NP, D), jnp.float32),
              jax.ShapeDtypeStruct((NC, NP, 8), jnp.float32)),
    compiler_params=_SC_PARAMS,
    scratch_types=[
        pltpu.VMEM((EB,), jnp.int32),
        pltpu.VMEM((EB // 2, D), jnp.float32),
        pltpu.VMEM((NCHUNK, CH), jnp.int32),
        pltpu.VMEM((CH, 8), jnp.float32),
        pltpu.VMEM_SHARED((NP, 8), jnp.float32),
        pltpu.SemaphoreType.DMA,
    ],
)
def _sc_emb_deg(emb_hbm, xp_hbm, dstb_hbm, ones_hbm, z16_hbm,
                h_hbm, degp_hbm,
                xidx_v, hrows_v, dst_v, ones_v, acc16, sem):
    cid = lax.axis_index("c")
    sid = lax.axis_index("s")
    wid = sid * NC + cid
    base = wid * EB

    pltpu.sync_copy(xp_hbm.at[pl.ds(base, EB)], xidx_v)
    for half in range(2):
        off = half * (EB // 2)
        pltpu.async_copy(emb_hbm.at[xidx_v.at[pl.ds(off, EB // 2)]],
                         hrows_v, sem).wait()
        pltpu.sync_copy(hrows_v, h_hbm.at[pl.ds(base + off, EB // 2)])

    pltpu.sync_copy(ones_hbm, ones_v)
    pltpu.sync_copy(dstb_hbm.at[wid], dst_v)
    pltpu.sync_copy(z16_hbm, acc16.at[pl.ds(sid * RPT, RPT)])
    plsc.subcore_barrier()

    def deg_body(j, carry):
        pltpu.sync_copy(ones_v, acc16.at[dst_v.at[j]], add=True)
        return carry

    lax.fori_loop(0, NCHUNK, deg_body, 0)
    plsc.subcore_barrier()
    pltpu.sync_copy(acc16.at[pl.ds(sid * RPT, RPT)],
                    degp_hbm.at[cid].at[pl.ds(sid * RPT, RPT)])


@functools.partial(
    pl.kernel, mesh=_mesh(),
    out_type=(jax.ShapeDtypeStruct((NC, NP, 32), jnp.float32),
              jax.ShapeDtypeStruct((NC, NP, 32), jnp.float32)),
    compiler_params=_SC_PARAMS,
    scratch_types=[
        pltpu.VMEM((2, 8, 2, CH), jnp.int32),
        pltpu.VMEM((NBUF, CH, 32), jnp.float32),
        pltpu.VMEM_SHARED((NP, 32), jnp.float32),
        pltpu.SemaphoreType.DMA,
        pltpu.SemaphoreType.DMA,
        pltpu.SemaphoreType.DMA,
        pltpu.SemaphoreType.DMA,
        pltpu.SemaphoreType.DMA,
        pltpu.SemaphoreType.DMA,
        pltpu.SemaphoreType.DMA,
        pltpu.SemaphoreType.DMA,
        pltpu.SemaphoreType.DMA,
    ],
)
def _sc_edge_pass(hsa_hbm, hsb_hbm, edges_hbm, z32_hbm,
                  oa_hbm, ob_hbm,
                  iblk, rbuf, acc, isem,
                  g0, g1, g2, g3, t0, t1, t2, t3):
    cid = lax.axis_index("c")
    sid = lax.axis_index("s")
    gsems = (g0, g1, g2, g3)
    ssems = (t0, t1, t2, t3)
    nblk = jnp.where(cid == CF, NBF, NBS)
    nchunk = nblk * 8
    base = jnp.where(cid == CF, 0, NBF)

    for p in range(2):
        hs_hbm = hsa_hbm if p == 0 else hsb_hbm
        out_hbm = oa_hbm if p == 0 else ob_hbm

        pltpu.sync_copy(z32_hbm, acc.at[pl.ds(sid * RPT, RPT)])
        pltpu.sync_copy(edges_hbm.at[sid, base], iblk.at[0])
        plsc.subcore_barrier()

        for b in range(2):
            pltpu.async_copy(hs_hbm.at[iblk.at[0, b, 0]], rbuf.at[b], gsems[b])
        pltpu.async_copy(edges_hbm.at[sid, base + 1], iblk.at[1], isem)

        def outer(s, carry):
            cur = s % 2
            nxtb = (s + 1) % 2
            for j in range(8):
                b = j % NBUF
                bn = (j + 2) % NBUF
                g = s * 8 + j
                pltpu.make_async_copy(hs_hbm.at[iblk.at[cur, j, 0]],
                                      rbuf.at[b], gsems[b]).wait()
                pltpu.async_copy(rbuf.at[b], acc.at[iblk.at[cur, j, 1]],
                                 ssems[b], add=True)
                if j == 4:
                    @pl.when(s + 1 < nblk)
                    def _():
                        pltpu.make_async_copy(edges_hbm.at[sid, base],
                                              iblk.at[nxtb], isem).wait()
                @pl.when(g >= 2)
                def _():
                    pltpu.make_async_copy(rbuf.at[bn],
                                          acc.at[iblk.at[cur, j, 1]],
                                          ssems[bn]).wait()
                if j < 6:
                    @pl.when(g + 2 < nchunk)
                    def _():
                        pltpu.async_copy(hs_hbm.at[iblk.at[cur, j + 2, 0]],
                                         rbuf.at[bn], gsems[bn])
                else:
                    @pl.when(g + 2 < nchunk)
                    def _():
                        pltpu.async_copy(hs_hbm.at[iblk.at[nxtb, j - 6, 0]],
                                         rbuf.at[bn], gsems[bn])
            @pl.when(s + 2 < nblk)
            def _():
                pltpu.async_copy(edges_hbm.at[sid, base + s + 2],
                                 iblk.at[cur], isem)
            return carry

        lax.fori_loop(0, nblk, outer, 0)
        for b in (2, 3):
            pltpu.make_async_copy(rbuf.at[b], acc.at[iblk.at[0, 0, 1]],
                                  ssems[b]).wait()
        plsc.subcore_barrier()
        pltpu.sync_copy(acc.at[pl.ds(sid * RPT, RPT)],
                        out_hbm.at[cid].at[pl.ds(sid * RPT, RPT)])
        plsc.subcore_barrier()


def _tc_scale_kernel(d0_ref, d1_ref, h_ref, hsa_ref, hsb_ref, dinv_ref):
    deg = d0_ref[0] + d1_ref[0] + 1.0
    dinv8 = lax.rsqrt(jnp.maximum(deg, 1.0))
    dinv16 = jnp.concatenate([dinv8, dinv8], axis=1)
    dinv32 = jnp.concatenate([dinv16, dinv16], axis=1)
    dinv64 = jnp.concatenate([dinv32, dinv32], axis=1)
    hs = h_ref[...] * dinv64
    hsa_ref[...] = hs[:, :32]
    hsb_ref[...] = hs[:, 32:]
    dinv_ref[...] = dinv32


def _tc_scale(degp, h):
    grid = NP // RPT
    return pl.pallas_call(
        _tc_scale_kernel,
        grid=(grid,),
        in_specs=[
            pl.BlockSpec((1, RPT, 8), lambda i: (0, i, 0)),
            pl.BlockSpec((1, RPT, 8), lambda i: (1, i, 0)),
            pl.BlockSpec((RPT, D), lambda i: (i, 0)),
        ],
        out_specs=[
            pl.BlockSpec((RPT, 32), lambda i: (i, 0)),
            pl.BlockSpec((RPT, 32), lambda i: (i, 0)),
            pl.BlockSpec((RPT, 32), lambda i: (i, 0)),
        ],
        out_shape=[
            jax.ShapeDtypeStruct((NP, 32), jnp.float32),
            jax.ShapeDtypeStruct((NP, 32), jnp.float32),
            jax.ShapeDtypeStruct((NP, 32), jnp.float32),
        ],
    )(degp, degp, h)


def _tc_layer_kernel(a0_ref, a1_ref, b0_ref, b1_ref, hsa_ref, hsb_ref,
                     dinv_ref, w_ref, brow_ref, hs1a_ref, hs1b_ref):
    dinv = dinv_ref[...]
    ta = dinv * (a0_ref[0] + a1_ref[0] + hsa_ref[...])
    tb = dinv * (b0_ref[0] + b1_ref[0] + hsb_ref[...])
    t = jnp.concatenate([ta, tb], axis=1)
    h1 = jnp.maximum(jnp.dot(t, w_ref[...],
                             preferred_element_type=jnp.float32)
                     + brow_ref[...], 0.0)
    hs1a_ref[...] = h1[:, :32] * dinv
    hs1b_ref[...] = h1[:, 32:] * dinv


def _tc_layer(oa, ob, hsa, hsb, dinv, W, b):
    grid = NP // RPT
    row = pl.BlockSpec((RPT, 32), lambda i: (i, 0))
    part = pl.BlockSpec((1, RPT, 32), lambda i: (0, i, 0))
    part1 = pl.BlockSpec((1, RPT, 32), lambda i: (1, i, 0))
    return pl.pallas_call(
        _tc_layer_kernel,
        grid=(grid,),
        in_specs=[part, part1, part, part1, row, row, row,
                  pl.BlockSpec((D, D), lambda i: (0, 0)),
                  pl.BlockSpec((1, D), lambda i: (0, 0))],
        out_specs=[row, row],
        out_shape=[
            jax.ShapeDtypeStruct((NP, 32), jnp.float32),
            jax.ShapeDtypeStruct((NP, 32), jnp.float32),
        ],
    )(oa, oa, ob, ob, hsa, hsb, dinv, W, b.reshape(1, D))


def _tc_final_kernel(a0_ref, a1_ref, b0_ref, b1_ref, hsa_ref, hsb_ref,
                     dinv_ref, w_ref, brow_ref, batch_ref, wc_ref, bc_ref,
                     out_ref, psum, pcnt):
    i = pl.program_id(0)
    dinv = dinv_ref[...]
    ta = dinv * (a0_ref[0] + a1_ref[0] + hsa_ref[...])
    tb = dinv * (b0_ref[0] + b1_ref[0] + hsb_ref[...])
    t = jnp.concatenate([ta, tb], axis=1)
    h2 = jnp.maximum(jnp.dot(t, w_ref[...],
                             preferred_element_type=jnp.float32)
                     + brow_ref[...], 0.0)
    gid = batch_ref[0]
    gids = jax.lax.broadcast_in_dim(gid, (G, RPT), (0, 1))
    rows = jax.lax.broadcasted_iota(jnp.int32, (G, RPT), 0)
    onehot = jnp.where(gids == rows, 1.0, 0.0)

    @pl.when(i == 0)
    def _():
        psum[...] = jnp.zeros_like(psum)
        pcnt[...] = jnp.zeros_like(pcnt)

    psum[...] += jnp.dot(onehot, h2, preferred_element_type=jnp.float32)
    pcnt[...] += jnp.dot(onehot, jnp.ones((RPT, D), jnp.float32),
                         preferred_element_type=jnp.float32)

    @pl.when(i == pl.num_programs(0) - 1)
    def _():
        pooled = psum[...] / jnp.maximum(pcnt[...], 1.0)
        out_ref[...] = (jnp.dot(pooled, wc_ref[...],
                                preferred_element_type=jnp.float32)
                        + bc_ref[...])


def _tc_final(oa, ob, hsa, hsb, dinv, W, b, batch3, Wc, bc):
    grid = NP // RPT
    row = pl.BlockSpec((RPT, 32), lambda i: (i, 0))
    part = pl.BlockSpec((1, RPT, 32), lambda i: (0, i, 0))
    part1 = pl.BlockSpec((1, RPT, 32), lambda i: (1, i, 0))
    return pl.pallas_call(
        _tc_final_kernel,
        grid=(grid,),
        in_specs=[part, part1, part, part1, row, row, row,
                  pl.BlockSpec((D, D), lambda i: (0, 0)),
                  pl.BlockSpec((1, D), lambda i: (0, 0)),
                  pl.BlockSpec((1, 1, RPT), lambda i: (i, 0, 0)),
                  pl.BlockSpec((D, 2), lambda i: (0, 0)),
                  pl.BlockSpec((1, 2), lambda i: (0, 0))],
        out_specs=pl.BlockSpec((G, 2), lambda i: (0, 0)),
        out_shape=jax.ShapeDtypeStruct((G, 2), jnp.float32),
        scratch_shapes=[pltpu.VMEM((G, D), jnp.float32),
                        pltpu.VMEM((G, D), jnp.float32)],
    )(oa, oa, ob, ob, hsa, hsb, dinv, W, b.reshape(1, D), batch3,
      Wc, bc.reshape(1, 2))


def kernel(x, edge_index, batch, emb, W1, b1, W2, b2, Wc, bc):
    xp = jnp.concatenate([x, jnp.zeros((NP - N,), jnp.int32)])
    src_pad = jnp.concatenate(
        [edge_index[0], jnp.zeros((EP - E,), jnp.int32)])
    dst_pad = jnp.concatenate(
        [edge_index[1], jnp.full((EP - E,), TRASH, jnp.int32)])
    nfe = NS * NBF * 8 * CH
    ef = jnp.concatenate(
        [src_pad[:nfe].reshape(NS, NBF, 8, 1, CH),
         dst_pad[:nfe].reshape(NS, NBF, 8, 1, CH)], axis=3)
    es = jnp.concatenate(
        [src_pad[nfe:].reshape(NS, NBS, 8, 1, CH),
         dst_pad[nfe:].reshape(NS, NBS, 8, 1, CH)], axis=3)
    edges5 = jnp.concatenate([ef, es], axis=1)
    dstb = dst_pad.reshape(NW, NCHUNK, CH)
    batch3 = jnp.concatenate(
        [batch, jnp.full((NP - N,), G, jnp.int32)]
    ).reshape(NP // RPT, 1, RPT)

    ones16 = jnp.ones((CH, 8), jnp.float32)
    z16 = jnp.zeros((RPT, 8), jnp.float32)
    z32 = jnp.zeros((RPT, 32), jnp.float32)

    h, degp = _sc_emb_deg(emb, xp, dstb, ones16, z16)
    hsa, hsb, dinv = _tc_scale(degp, h)
    oa1, ob1 = _sc_edge_pass(hsa, hsb, edges5, z32)
    hs1a, hs1b = _tc_layer(oa1, ob1, hsa, hsb, dinv, W1, b1)
    oa2, ob2 = _sc_edge_pass(hs1a, hs1b, edges5, z32)
    return _tc_final(oa2, ob2, hs1a, hs1b, dinv, W2, b2, batch3, Wc, bc)
